# baseline (speedup 1.0000x reference)
import functools

import jax
import jax.numpy as jnp
from jax import lax
from jax.experimental import pallas as pl
from jax.experimental.pallas import tpu as pltpu
from jax.experimental.pallas import tpu_sc as plsc

BATCH = 16384
HIST = 200
EMBED = 32
TOTAL = BATCH * HIST
NUM_CORES = 2
NUM_SUBCORES = 16
NW = NUM_CORES * NUM_SUBCORES
BWIN = BATCH // NW
CHUNK = BWIN
NCHUNK = HIST
NBG = 3
NBS = 2
LOOK = 2
OUT_ROWS = TOTAL * EMBED // 128


def _embedding_body(table_hbm, idx_hbm, out_hbm, idx_v, rows_v, trans_v,
                    gsem, ssem):
    wid = lax.axis_index("s") * NUM_CORES + lax.axis_index("c")
    bbase = wid * BWIN
    iota16 = lax.iota(jnp.int32, 16)

    def issue_gather(h, b):
        pltpu.sync_copy(idx_hbm.at[pl.ds(h * BATCH + bbase, CHUNK)],
                        idx_v.at[b])
        pltpu.async_copy(table_hbm.at[idx_v.at[b]], rows_v.at[b], gsem.at[b])

    def wait_gather(b):
        pltpu.make_async_copy(
            table_hbm.at[idx_v.at[b]], rows_v.at[b], gsem.at[b]).wait()

    def issue_store(h, bt):
        for dblk in range(4):
            rowbase = h * 4096 + dblk * 1024 + wid * 32
            pltpu.async_copy(
                trans_v.at[bt, pl.ds(dblk * 40, 32), pl.ds(0, 128)],
                out_hbm.at[pl.ds(rowbase, 32), :], ssem.at[bt])

    def wait_store(bt):
        for dblk in range(4):
            pltpu.make_async_copy(
                trans_v.at[bt, pl.ds(dblk * 40, 32), pl.ds(0, 128)],
                out_hbm.at[pl.ds(0, 32), :], ssem.at[bt]).wait()

    row2c0 = ((iota16 >> 3) * 40 + (iota16 & 7)).astype(jnp.int32)
    row2c1 = row2c0 + 80

    def transpose(b, bt):
        trans = trans_v.at[bt]

        @plsc.parallel_loop(0, CHUNK, step=8, unroll=8)
        def _(r0):
            bblk = r0 >> 7
            roff = bblk * 8
            rv0 = row2c0 + roff
            rv1 = row2c1 + roff
            col0 = jnp.full((16,), r0 & 127, jnp.int32)
            for k in range(8):
                r = r0 + k
                col = col0 + k
                v0 = rows_v[b, r, pl.ds(0, 16)]
                plsc.store_scatter(trans, [rv0, col], v0)
                v1 = rows_v[b, r, pl.ds(16, 16)]
                plsc.store_scatter(trans, [rv1, col], v1)

    for n in range(LOOK):
        issue_gather(n, n)

    for c in range(2):
        issue_gather(c + LOOK, (c + LOOK) % NBG)
        wait_gather(c % NBG)
        transpose(c % NBG, c % NBS)
        issue_store(c, c % NBS)

    def group(g, carry):
        for k in range(6):
            c = 2 + g * 6 + k
            b = (2 + k) % NBG
            bt = k % NBS
            wait_store(bt)
            issue_gather(c + LOOK, (2 + k + LOOK) % NBG)
            wait_gather(b)
            transpose(b, bt)
            issue_store(c, bt)
        return carry

    lax.fori_loop(0, (NCHUNK - 2 - 6) // 6, group, 0)

    for c in range(NCHUNK - 6, NCHUNK):
        b = c % NBG
        bt = c % NBS
        wait_store(bt)
        if c + LOOK < NCHUNK:
            issue_gather(c + LOOK, (c + LOOK) % NBG)
        wait_gather(b)
        transpose(b, bt)
        issue_store(c, bt)

    for bt in range(NBS):
        wait_store(bt)


def kernel(indices, W):
    flat = indices.T.reshape(TOTAL).astype(jnp.int32)
    mesh = plsc.VectorSubcoreMesh(core_axis_name="c", subcore_axis_name="s")
    run = functools.partial(
        pl.kernel,
        mesh=mesh,
        out_type=jax.ShapeDtypeStruct((OUT_ROWS, 128), jnp.float32),
        scratch_types=[
            pltpu.VMEM((NBG, CHUNK), jnp.int32),
            pltpu.VMEM((NBG, CHUNK, EMBED), jnp.float32),
            pltpu.VMEM((NBS, 160, 129), jnp.float32),
            pltpu.SemaphoreType.DMA((NBG,)),
            pltpu.SemaphoreType.DMA((NBS,)),
        ],
        compiler_params=pltpu.CompilerParams(use_tc_tiling_on_sc=False,
                                             needs_layout_passes=False),
    )(_embedding_body)
    out = run(W, flat)
    out = out.reshape(HIST, 4, BATCH // 128, 8, 128)
    out = out.transpose(2, 4, 0, 1, 3)
    return out.reshape(BATCH, HIST, EMBED)


# speedup vs baseline: 1.2434x; 1.2434x over previous
import functools

import jax
import jax.numpy as jnp
from jax import lax
from jax.experimental import pallas as pl
from jax.experimental.pallas import tpu as pltpu
from jax.experimental.pallas import tpu_sc as plsc

BATCH = 16384
HIST = 200
EMBED = 32
TOTAL = BATCH * HIST
NUM_CORES = 2
NUM_SUBCORES = 16
NW = NUM_CORES * NUM_SUBCORES
BWIN = BATCH // NW
CHUNK = BWIN
NCHUNK = HIST
NBG = 3
NBS = 2
LOOK = 2
OUT_ROWS = TOTAL * EMBED // 128


def _embedding_body(table_hbm, idx_hbm, out_hbm, idx_v, rows_v, trans_v,
                    gsem, ssem):
    wid = lax.axis_index("s") * NUM_CORES + lax.axis_index("c")
    bbase = wid * BWIN
    iota16 = lax.iota(jnp.int32, 16)

    def issue_gather(h, b):
        pltpu.sync_copy(idx_hbm.at[pl.ds(h * BATCH + bbase, CHUNK)],
                        idx_v.at[b])
        pltpu.async_copy(table_hbm.at[idx_v.at[b]], rows_v.at[b], gsem.at[b])

    def wait_gather(b):
        pltpu.make_async_copy(
            table_hbm.at[idx_v.at[b]], rows_v.at[b], gsem.at[b]).wait()

    def issue_store(h, bt):
        for dblk in range(4):
            rowbase = h * 4096 + dblk * 1024 + wid * 32
            pltpu.async_copy(
                trans_v.at[bt, pl.ds(dblk * 40, 32), pl.ds(0, 128)],
                out_hbm.at[pl.ds(rowbase, 32), :], ssem.at[bt])

    def wait_store(bt):
        for dblk in range(4):
            pltpu.make_async_copy(
                trans_v.at[bt, pl.ds(dblk * 40, 32), pl.ds(0, 128)],
                out_hbm.at[pl.ds(0, 32), :], ssem.at[bt]).wait()

    row2c0 = ((iota16 >> 3) * 40 + (iota16 & 7)).astype(jnp.int32)
    row2c1 = row2c0 + 80

    def transpose(b, bt):
        trans = trans_v.at[bt]

        @plsc.parallel_loop(0, CHUNK, step=8, unroll=4)
        def _(r0):
            bblk = r0 >> 7
            roff = bblk * 8
            rv0 = row2c0 + roff
            rv1 = row2c1 + roff
            col0 = jnp.full((16,), r0 & 127, jnp.int32)
            for k in range(8):
                r = r0 + k
                col = col0 + k
                v0 = rows_v[b, r, pl.ds(0, 16)]
                plsc.store_scatter(trans, [rv0, col], v0)
                v1 = rows_v[b, r, pl.ds(16, 16)]
                plsc.store_scatter(trans, [rv1, col], v1)

    for n in range(LOOK):
        issue_gather(n, n)

    for c in range(2):
        issue_gather(c + LOOK, (c + LOOK) % NBG)
        wait_gather(c % NBG)
        transpose(c % NBG, c % NBS)
        issue_store(c, c % NBS)

    def group(g, carry):
        for k in range(6):
            c = 2 + g * 6 + k
            b = (2 + k) % NBG
            bt = k % NBS
            wait_store(bt)
            issue_gather(c + LOOK, (2 + k + LOOK) % NBG)
            wait_gather(b)
            transpose(b, bt)
            issue_store(c, bt)
        return carry

    lax.fori_loop(0, (NCHUNK - 2 - 6) // 6, group, 0)

    for c in range(NCHUNK - 6, NCHUNK):
        b = c % NBG
        bt = c % NBS
        wait_store(bt)
        if c + LOOK < NCHUNK:
            issue_gather(c + LOOK, (c + LOOK) % NBG)
        wait_gather(b)
        transpose(b, bt)
        issue_store(c, bt)

    for bt in range(NBS):
        wait_store(bt)


def kernel(indices, W):
    flat = indices.T.reshape(TOTAL).astype(jnp.int32)
    mesh = plsc.VectorSubcoreMesh(core_axis_name="c", subcore_axis_name="s")
    run = functools.partial(
        pl.kernel,
        mesh=mesh,
        out_type=jax.ShapeDtypeStruct((OUT_ROWS, 128), jnp.float32),
        scratch_types=[
            pltpu.VMEM((NBG, CHUNK), jnp.int32),
            pltpu.VMEM((NBG, CHUNK, EMBED), jnp.float32),
            pltpu.VMEM((NBS, 160, 129), jnp.float32),
            pltpu.SemaphoreType.DMA((NBG,)),
            pltpu.SemaphoreType.DMA((NBS,)),
        ],
        compiler_params=pltpu.CompilerParams(use_tc_tiling_on_sc=False,
                                             needs_layout_passes=False),
    )(_embedding_body)
    out = run(W, flat)
    out = out.reshape(HIST, 4, BATCH // 128, 8, 128)
    out = out.transpose(2, 4, 0, 1, 3)
    return out.reshape(BATCH, HIST, EMBED)


# speedup vs baseline: 1.3275x; 1.0676x over previous
import functools

import jax
import jax.numpy as jnp
from jax import lax
from jax.experimental import pallas as pl
from jax.experimental.pallas import tpu as pltpu
from jax.experimental.pallas import tpu_sc as plsc

BATCH = 16384
HIST = 200
EMBED = 32
TOTAL = BATCH * HIST
NUM_CORES = 2
NUM_SUBCORES = 16
NW = NUM_CORES * NUM_SUBCORES
BWIN = BATCH // NW
CHUNK = BWIN
NCHUNK = HIST
NBG = 3
NBS = 2
LOOK = 2
OUT_ROWS = TOTAL * EMBED // 128


def _embedding_body(table_hbm, idx_hbm, out_hbm, idx_v, rows_v, trans_v,
                    gsem, ssem):
    wid = lax.axis_index("s") * NUM_CORES + lax.axis_index("c")
    bbase = wid * BWIN
    iota16 = lax.iota(jnp.int32, 16)

    def issue_gather(h, b):
        pltpu.sync_copy(idx_hbm.at[pl.ds(h * BATCH + bbase, CHUNK)],
                        idx_v.at[b])
        pltpu.async_copy(table_hbm.at[idx_v.at[b]], rows_v.at[b], gsem.at[b])

    def wait_gather(b):
        pltpu.make_async_copy(
            table_hbm.at[idx_v.at[b]], rows_v.at[b], gsem.at[b]).wait()

    def issue_store(h, bt):
        for dblk in range(4):
            rowbase = h * 4096 + dblk * 1024 + wid * 32
            pltpu.async_copy(
                trans_v.at[bt, pl.ds(dblk * 40, 32), pl.ds(0, 128)],
                out_hbm.at[pl.ds(rowbase, 32), :], ssem.at[bt])

    def wait_store(bt):
        for dblk in range(4):
            pltpu.make_async_copy(
                trans_v.at[bt, pl.ds(dblk * 40, 32), pl.ds(0, 128)],
                out_hbm.at[pl.ds(0, 32), :], ssem.at[bt]).wait()

    row2c0 = ((iota16 >> 3) * 40 + (iota16 & 7)).astype(jnp.int32)
    row2c1 = row2c0 + 80

    def transpose(b, bt):
        trans = trans_v.at[bt]

        @plsc.parallel_loop(0, CHUNK, step=8, unroll=4)
        def _(r0):
            bblk = r0 >> 7
            roff = bblk * 8
            rv0 = row2c0 + roff
            rv1 = row2c1 + roff
            col0 = jnp.full((16,), r0 & 127, jnp.int32)
            for k in range(8):
                r = r0 + k
                col = col0 + k
                v0 = rows_v[b, r, pl.ds(0, 16)]
                plsc.store_scatter(trans, [rv0, col], v0)
                v1 = rows_v[b, r, pl.ds(16, 16)]
                plsc.store_scatter(trans, [rv1, col], v1)

    for n in range(LOOK):
        issue_gather(n, n)

    for c in range(2):
        issue_gather(c + LOOK, (c + LOOK) % NBG)
        wait_gather(c % NBG)
        transpose(c % NBG, c % NBS)
        issue_store(c, c % NBS)

    def group(g, carry):
        for k in range(6):
            c = 2 + g * 6 + k
            b = (2 + k) % NBG
            bt = k % NBS
            wait_store(bt)
            issue_gather(c + LOOK, (2 + k + LOOK) % NBG)
            wait_gather(b)
            transpose(b, bt)
            issue_store(c, bt)
        return carry

    lax.fori_loop(0, (NCHUNK - 2 - 6) // 6, group, 0)

    for c in range(NCHUNK - 6, NCHUNK):
        b = c % NBG
        bt = c % NBS
        wait_store(bt)
        if c + LOOK < NCHUNK:
            issue_gather(c + LOOK, (c + LOOK) % NBG)
        wait_gather(b)
        transpose(b, bt)
        issue_store(c, bt)

    for bt in range(NBS):
        wait_store(bt)


VOCABN = 1000000
NFULL = VOCABN // 128
NTAIL = VOCABN - NFULL * 128
WPB = 245


def _w_transpose_body(wt_hbm, tail_hbm, out_hbm, stag0, stag1, trans0,
                      trans1, tail_v, gsem, ssem):
    stag_b = (stag0, stag1)
    trans_b = (trans0, trans1)
    wid = lax.axis_index("s") * NUM_CORES + lax.axis_index("c")
    iota = lax.iota(jnp.int32, 16)
    iota_hi = iota + 16
    cb0 = wid * WPB

    def issue_stage(i, s):
        cb = cb0 + i
        pltpu.async_copy(wt_hbm.at[:, pl.ds(cb * 128, 128)],
                         stag_b[s].at[:, pl.ds(0, 128)], gsem.at[s])

    def wait_stage(i, s):
        pltpu.make_async_copy(wt_hbm.at[:, pl.ds(0, 128)],
                              stag_b[s].at[:, pl.ds(0, 128)],
                              gsem.at[s]).wait()

    def issue_store(i, s):
        cb = cb0 + i
        pltpu.async_copy(trans_b[s],
                         out_hbm.at[pl.ds(cb * 4096, 4096)], ssem.at[s])

    def wait_store(i, s):
        pltpu.make_async_copy(trans_b[s], out_hbm.at[pl.ds(0, 4096)],
                              ssem.at[s]).wait()

    def transpose(s):
        stag = stag_b[s]
        trans = trans_b[s]

        @plsc.parallel_loop(0, 128, step=4, unroll=4)
        def _(vr0):
            for k in range(4):
                vr = vr0 + k
                colv = jnp.full((16,), vr, jnp.int32)
                v0 = plsc.load_gather(stag, [iota, colv])
                trans[pl.ds(vr * 32, 16)] = v0
                v1 = plsc.load_gather(stag, [iota_hi, colv])
                trans[pl.ds(vr * 32 + 16, 16)] = v1

    nmine = jnp.minimum(WPB, jnp.maximum(0, NFULL - cb0))

    def valid(i):
        return i < nmine

    @pl.when(valid(0))
    def _():
        issue_stage(0, 0)

    def pair(g, carry):
        for k in range(2):
            i = g * 2 + k
            s = k

            @pl.when(valid(i + 1))
            def _():
                issue_stage(i + 1, (k + 1) % 2)

            @pl.when(valid(i))
            def _():
                wait_stage(i, s)

                @pl.when(i >= 2)
                def _():
                    wait_store(i - 2, s)

                transpose(s)
                issue_store(i, s)

        return carry

    lax.fori_loop(0, (WPB + 1) // 2, pair, 0)

    wait_store(nmine - 2, 1)
    wait_store(nmine - 1, 0)

    @pl.when(wid == 0)
    def _():
        pltpu.sync_copy(tail_hbm, tail_v)
        pltpu.sync_copy(tail_v, out_hbm.at[pl.ds(NFULL * 4096, NTAIL * 32)])


def kernel(indices, W):
    flat = indices.T.reshape(TOTAL).astype(jnp.int32)
    mesh = plsc.VectorSubcoreMesh(core_axis_name="c", subcore_axis_name="s")
    wrun = functools.partial(
        pl.kernel,
        mesh=mesh,
        out_type=jax.ShapeDtypeStruct((VOCABN * EMBED,), jnp.float32),
        scratch_types=[
            pltpu.VMEM((32, 129), jnp.float32),
            pltpu.VMEM((32, 129), jnp.float32),
            pltpu.VMEM((4096,), jnp.float32),
            pltpu.VMEM((4096,), jnp.float32),
            pltpu.VMEM((NTAIL * 32,), jnp.float32),
            pltpu.SemaphoreType.DMA((2,)),
            pltpu.SemaphoreType.DMA((2,)),
        ],
        compiler_params=pltpu.CompilerParams(use_tc_tiling_on_sc=True,
                                             needs_layout_passes=False),
    )(_w_transpose_body)
    tail = W[NFULL * 128:].reshape(NTAIL * EMBED)
    w_lin = wrun(W.T, tail).reshape(VOCABN, EMBED)
    run = functools.partial(
        pl.kernel,
        mesh=mesh,
        out_type=jax.ShapeDtypeStruct((OUT_ROWS, 128), jnp.float32),
        scratch_types=[
            pltpu.VMEM((NBG, CHUNK), jnp.int32),
            pltpu.VMEM((NBG, CHUNK, EMBED), jnp.float32),
            pltpu.VMEM((NBS, 160, 129), jnp.float32),
            pltpu.SemaphoreType.DMA((NBG,)),
            pltpu.SemaphoreType.DMA((NBS,)),
        ],
        compiler_params=pltpu.CompilerParams(use_tc_tiling_on_sc=False,
                                             needs_layout_passes=False),
    )(_embedding_body)
    out = run(w_lin, flat)
    out = out.reshape(HIST, 4, BATCH // 128, 8, 128)
    out = out.transpose(2, 4, 0, 1, 3)
    return out.reshape(BATCH, HIST, EMBED)
